# Initial kernel scaffold; baseline (speedup 1.0000x reference)
import jax
import jax.numpy as jnp
from jax.experimental import pallas as pl


def kernel(time_vector, constant_offset, linear_trend, seasonal_amplitudes, seasonal_phases, neighbor_indices, neighbor_weights, min_bounds, max_bounds):
    raise NotImplementedError("write your pallas kernel here")


# speedup vs baseline: 123.9406x; 123.9406x over previous
import functools

import jax
import jax.numpy as jnp
import numpy as np
from jax import lax
from jax.experimental import pallas as pl
from jax.experimental.pallas import tpu as pltpu
from jax.experimental.pallas import tpu_sc as plsc

_N = 100000
_K = 16
_T = 256
_NW = 32
_PW = _N // _NW
_BLK = 125
_GS = 5
_BN = 2000


def _lane_bcast(v, k):
    idx = jnp.full((16, 1), k, dtype=jnp.int32)
    dn = lax.GatherDimensionNumbers(
        offset_dims=(), collapsed_slice_dims=(0,), start_index_map=(0,))
    return lax.gather(v, idx, dn, (1,),
                      mode=lax.GatherScatterMode.PROMISE_IN_BOUNDS)


def _trig_body(ph_ref, cos_ref, sin_ref):
    x = ph_ref[...]
    cos_ref[...] = jnp.cos(x)
    sin_ref[...] = jnp.sin(x)


def _sc_body(table, idxf, wf, out, idx_v, w_v, g_v, out_v, sem):
    wid = lax.axis_index("s") * 2 + lax.axis_index("c")

    def blk(b, carry):
        base = wid * _PW + b * _BLK
        pltpu.sync_copy(idxf.at[pl.ds(base * _K, _BLK * _K)], idx_v)
        pltpu.sync_copy(wf.at[pl.ds(base * _K, _BLK * _K)], w_v)
        handles = []
        for c in range(_BLK // _GS):
            handles.append(pltpu.async_copy(
                table.at[idx_v.at[pl.ds(c * _GS * _K, _GS * _K)]],
                g_v.at[pl.ds(c * _GS * _K, _GS * _K)], sem))
        for h in handles:
            h.wait()

        def st(s, carry2):
            w_row = w_v[pl.ds(s * _K, _K)]
            r0 = s * _K
            acc = _lane_bcast(w_row, 0) * g_v[r0]
            for k in range(1, _K):
                acc = acc + _lane_bcast(w_row, k) * g_v[r0 + k]
            out_v[pl.ds(s * _K, _K)] = acc
            return carry2

        lax.fori_loop(0, _BLK, st, 0)
        pltpu.sync_copy(out_v, out.at[pl.ds(base * _K, _BLK * _K)])
        return carry

    lax.fori_loop(0, _PW // _BLK, blk, 0)


_sc_gather = functools.partial(
    pl.kernel,
    mesh=plsc.VectorSubcoreMesh(core_axis_name="c", subcore_axis_name="s"),
    out_type=jax.ShapeDtypeStruct((_N * 16,), jnp.float32),
    scratch_types=[
        pltpu.VMEM((_BLK * _K,), jnp.int32),
        pltpu.VMEM((_BLK * _K,), jnp.float32),
        pltpu.VMEM((_BLK * _K, 16), jnp.float32),
        pltpu.VMEM((_BLK * 16,), jnp.float32),
        pltpu.SemaphoreType.DMA,
    ],
    compiler_params=pltpu.CompilerParams(use_tc_tiling_on_sc=False),
)(_sc_body)


def _syn_body(t_ref, a_ref, x_ref, b_ref, o_ref):
    T = t_ref[...]
    A = a_ref[...]
    X = x_ref[...]
    B = b_ref[...]
    amp = T[:, 0:4]
    sm = jnp.clip(0.85 * amp + 0.15 * A[:, 0:4], X[:, 0:4], X[:, 4:8])
    re = 0.9 * T[:, 4:8] + 0.1 * A[:, 4:8]
    im = 0.9 * T[:, 8:12] + 0.1 * A[:, 8:12]
    inv = lax.rsqrt(re * re + im * im)
    csin = sm * re * inv
    ccos = sm * im * inv
    C = jnp.concatenate(
        [csin, ccos, X[:, 8:10], jnp.zeros((_BN, 6), jnp.float32)], axis=1)
    o_ref[...] = lax.dot_general(
        C, B, (((1,), (0,)), ((), ())),
        preferred_element_type=jnp.float32, precision=lax.Precision.HIGHEST)


def kernel(time_vector, constant_offset, linear_trend, seasonal_amplitudes,
           seasonal_phases, neighbor_indices, neighbor_weights, min_bounds,
           max_bounds):
    f32 = jnp.float32
    ph_flat = seasonal_phases.astype(f32).reshape(_N * 4 // 128, 128)
    cosv, sinv = pl.pallas_call(
        _trig_body,
        out_shape=[jax.ShapeDtypeStruct(ph_flat.shape, f32)] * 2,
    )(ph_flat)

    amp = seasonal_amplitudes.astype(f32)
    tbl = jnp.concatenate(
        [amp, cosv.reshape(_N, 4), sinv.reshape(_N, 4),
         jnp.zeros((_N, 4), f32)], axis=1)
    aux = jnp.concatenate(
        [min_bounds.astype(f32), max_bounds.astype(f32),
         constant_offset.astype(f32)[:, None], linear_trend.astype(f32)[:, None],
         jnp.zeros((_N, 6), f32)], axis=1)

    idx_flat = neighbor_indices.astype(jnp.int32).reshape(_N * _K)
    w_flat = neighbor_weights.astype(f32).reshape(_N * _K)
    avg = _sc_gather(tbl, idx_flat, w_flat).reshape(_N, 16)

    tv = time_vector.astype(f32)
    freqs = jnp.array([4.0, 2.0, 1.0, 0.5], f32)
    ang = (2.0 * np.pi) * freqs[:, None] * tv[None, :]
    basis = jnp.concatenate(
        [jnp.sin(ang), jnp.cos(ang), jnp.ones((1, _T), f32), tv[None, :],
         jnp.zeros((6, _T), f32)], axis=0)

    nblk = _N // _BN
    out = pl.pallas_call(
        _syn_body,
        grid=(nblk,),
        in_specs=[
            pl.BlockSpec((_BN, 16), lambda i: (i, 0)),
            pl.BlockSpec((_BN, 16), lambda i: (i, 0)),
            pl.BlockSpec((_BN, 16), lambda i: (i, 0)),
            pl.BlockSpec((16, _T), lambda i: (0, 0)),
        ],
        out_specs=pl.BlockSpec((_BN, _T), lambda i: (i, 0)),
        out_shape=jax.ShapeDtypeStruct((_N, _T), f32),
    )(tbl, avg, aux, basis)
    return out


# speedup vs baseline: 128.2788x; 1.0350x over previous
import functools

import jax
import jax.numpy as jnp
import numpy as np
from jax import lax
from jax.experimental import pallas as pl
from jax.experimental.pallas import tpu as pltpu
from jax.experimental.pallas import tpu_sc as plsc

_N = 100000
_K = 16
_T = 256
_NW = 32
_PW = _N // _NW
_BLK = 125
_GS = 5
_BN = 2000


def _lane_bcast(v, k):
    idx = jnp.full((16, 1), k, dtype=jnp.int32)
    dn = lax.GatherDimensionNumbers(
        offset_dims=(), collapsed_slice_dims=(0,), start_index_map=(0,))
    return lax.gather(v, idx, dn, (1,),
                      mode=lax.GatherScatterMode.PROMISE_IN_BOUNDS)


def _trig_body(ph_ref, cos_ref, sin_ref):
    x = ph_ref[...]
    cos_ref[...] = jnp.cos(x)
    sin_ref[...] = jnp.sin(x)


def _sc_body(table, idxf, wf, out, idx_v, w_v, g_v, out_v, sem):
    wid = lax.axis_index("s") * 2 + lax.axis_index("c")

    def blk(b, carry):
        base = wid * _PW + b * _BLK
        pltpu.sync_copy(idxf.at[pl.ds(base * _K, _BLK * _K)], idx_v)
        pltpu.sync_copy(wf.at[pl.ds(base * _K, _BLK * _K)], w_v)
        handles = []
        for c in range(_BLK // _GS):
            handles.append(pltpu.async_copy(
                table.at[idx_v.at[pl.ds(c * _GS * _K, _GS * _K)]],
                g_v.at[pl.ds(c * _GS * _K, _GS * _K)], sem))
        for h in handles:
            h.wait()

        @plsc.parallel_loop(0, _BLK, unroll=2)
        def st(s):
            w_row = w_v[pl.ds(s * _K, _K)]
            r0 = s * _K
            acc = [_lane_bcast(w_row, k) * g_v[r0 + k] for k in range(4)]
            for k in range(4, _K):
                acc[k % 4] = acc[k % 4] + _lane_bcast(w_row, k) * g_v[r0 + k]
            out_v[pl.ds(s * _K, _K)] = (acc[0] + acc[1]) + (acc[2] + acc[3])
        pltpu.sync_copy(out_v, out.at[pl.ds(base * _K, _BLK * _K)])
        return carry

    lax.fori_loop(0, _PW // _BLK, blk, 0)


_sc_gather = functools.partial(
    pl.kernel,
    mesh=plsc.VectorSubcoreMesh(core_axis_name="c", subcore_axis_name="s"),
    out_type=jax.ShapeDtypeStruct((_N * 16,), jnp.float32),
    scratch_types=[
        pltpu.VMEM((_BLK * _K,), jnp.int32),
        pltpu.VMEM((_BLK * _K,), jnp.float32),
        pltpu.VMEM((_BLK * _K, 16), jnp.float32),
        pltpu.VMEM((_BLK * 16,), jnp.float32),
        pltpu.SemaphoreType.DMA,
    ],
    compiler_params=pltpu.CompilerParams(use_tc_tiling_on_sc=False),
)(_sc_body)


def _syn_body(t_ref, a_ref, lo_ref, hi_ref, ot_ref, b_ref, o_ref):
    T = t_ref[...]
    A = a_ref[...]
    B = b_ref[...]
    amp = T[:, 0:4]
    sm = jnp.clip(0.85 * amp + 0.15 * A[:, 0:4], lo_ref[...], hi_ref[...])
    re = 0.9 * T[:, 4:8] + 0.1 * A[:, 4:8]
    im = 0.9 * T[:, 8:12] + 0.1 * A[:, 8:12]
    inv = lax.rsqrt(re * re + im * im)
    csin = sm * re * inv
    ccos = sm * im * inv
    C = jnp.concatenate(
        [csin, ccos, ot_ref[...], jnp.zeros((_BN, 6), jnp.float32)], axis=1)
    o_ref[...] = lax.dot_general(
        C, B, (((1,), (0,)), ((), ())),
        preferred_element_type=jnp.float32, precision=lax.Precision.DEFAULT)


def kernel(time_vector, constant_offset, linear_trend, seasonal_amplitudes,
           seasonal_phases, neighbor_indices, neighbor_weights, min_bounds,
           max_bounds):
    f32 = jnp.float32
    ph_flat = seasonal_phases.astype(f32).reshape(_N * 4 // 128, 128)
    cosv, sinv = pl.pallas_call(
        _trig_body,
        out_shape=[jax.ShapeDtypeStruct(ph_flat.shape, f32)] * 2,
    )(ph_flat)

    amp = seasonal_amplitudes.astype(f32)
    tbl = jnp.concatenate(
        [amp, cosv.reshape(_N, 4), sinv.reshape(_N, 4),
         jnp.zeros((_N, 4), f32)], axis=1)
    idx_flat = neighbor_indices.astype(jnp.int32).reshape(_N * _K)
    w_flat = neighbor_weights.astype(f32).reshape(_N * _K)
    avg = _sc_gather(tbl, idx_flat, w_flat).reshape(_N, 16)

    tv = time_vector.astype(f32)
    freqs = jnp.array([4.0, 2.0, 1.0, 0.5], f32)
    ang = (2.0 * np.pi) * freqs[:, None] * tv[None, :]
    basis = jnp.concatenate(
        [jnp.sin(ang), jnp.cos(ang), jnp.ones((1, _T), f32), tv[None, :],
         jnp.zeros((6, _T), f32)], axis=0)

    nblk = _N // _BN
    out = pl.pallas_call(
        _syn_body,
        grid=(nblk,),
        in_specs=[
            pl.BlockSpec((_BN, 16), lambda i: (i, 0)),
            pl.BlockSpec((_BN, 16), lambda i: (i, 0)),
            pl.BlockSpec((_BN, 4), lambda i: (i, 0)),
            pl.BlockSpec((_BN, 4), lambda i: (i, 0)),
            pl.BlockSpec((_BN, 2), lambda i: (i, 0)),
            pl.BlockSpec((16, _T), lambda i: (0, 0)),
        ],
        out_specs=pl.BlockSpec((_BN, _T), lambda i: (i, 0)),
        out_shape=jax.ShapeDtypeStruct((_N, _T), f32),
    )(tbl, avg, min_bounds.astype(f32), max_bounds.astype(f32),
      jnp.stack([constant_offset.astype(f32), linear_trend.astype(f32)],
                axis=-1), basis)
    return out


# speedup vs baseline: 141.1348x; 1.1002x over previous
import functools

import jax
import jax.numpy as jnp
import numpy as np
from jax import lax
from jax.experimental import pallas as pl
from jax.experimental.pallas import tpu as pltpu
from jax.experimental.pallas import tpu_sc as plsc

_N = 100000
_K = 16
_T = 256
_NW = 32
_PW = _N // _NW
_BLK = 125
_GS = 5
_PR = _N // 32
_QR = _N // 8
_BN = 4096
_BQ = _BN // 8
_PB = _PR


def _station_slot(m):
    return (m // 8) * 128 + (m % 8) * 16


def _perm_consts():
    p1 = np.zeros((384, 512), np.float32)
    p2 = np.zeros((256, 512), np.float32)
    p3 = np.zeros((64, 512), np.float32)
    for l in range(128):
        m, i = l // 4, l % 4
        o = _station_slot(m)
        p1[l, o + i] = 1.0
        p1[128 + l, o + 4 + i] = 1.0
        p1[256 + l, o + 8 + i] = 1.0
        p2[l, o + i] = 1.0
        p2[128 + l, o + 4 + i] = 1.0
    for m in range(32):
        o = _station_slot(m)
        p3[2 * m, o + 13] = 1.0
        p3[2 * m + 1, o + 12] = 1.0
    return jnp.asarray(p1), jnp.asarray(p2), jnp.asarray(p3)


def _prep_body(amp_ref, ph_ref, mm_ref, ot_ref, p1_ref, p2_ref, p3_ref,
               tbl_ref, aux_ref):
    hi = lax.Precision.HIGHEST
    p = ph_ref[...]
    x1 = jnp.concatenate([amp_ref[...], jnp.cos(p), jnp.sin(p)], axis=1)
    tblw = (lax.dot_general(x1, p1_ref[...], (((1,), (0,)), ((), ())),
                            precision=hi)
            + lax.dot_general(ot_ref[...], p3_ref[...],
                              (((1,), (0,)), ((), ())), precision=hi))
    auxw = lax.dot_general(mm_ref[...], p2_ref[...], (((1,), (0,)), ((), ())),
                           precision=hi)
    tbl_ref[...] = tblw.reshape(4 * _PB, 128)
    aux_ref[...] = auxw.reshape(4 * _PB, 128)


def _lane_bcast(v, k):
    idx = jnp.full((16, 1), k, dtype=jnp.int32)
    dn = lax.GatherDimensionNumbers(
        offset_dims=(), collapsed_slice_dims=(0,), start_index_map=(0,))
    return lax.gather(v, idx, dn, (1,),
                      mode=lax.GatherScatterMode.PROMISE_IN_BOUNDS)


def _sc_body(table, idxf, wf, out, idx_v, w_v, g_v, out_v, sem):
    wid = lax.axis_index("s") * 2 + lax.axis_index("c")

    def blk(b, carry):
        base = wid * _PW + b * _BLK
        pltpu.sync_copy(idxf.at[pl.ds(base * _K, _BLK * _K)], idx_v)
        pltpu.sync_copy(wf.at[pl.ds(base * _K, _BLK * _K)], w_v)
        handles = []
        for c in range(_BLK // _GS):
            handles.append(pltpu.async_copy(
                table.at[idx_v.at[pl.ds(c * _GS * _K, _GS * _K)]],
                g_v.at[pl.ds(c * _GS * _K, _GS * _K)], sem))
        for h in handles:
            h.wait()

        @plsc.parallel_loop(0, _BLK, unroll=2)
        def st(s):
            w_row = w_v[pl.ds(s * _K, _K)]
            r0 = s * _K
            acc = [_lane_bcast(w_row, k) * g_v[r0 + k] for k in range(4)]
            for k in range(4, _K):
                acc[k % 4] = acc[k % 4] + _lane_bcast(w_row, k) * g_v[r0 + k]
            out_v[pl.ds(s * _K, _K)] = (acc[0] + acc[1]) + (acc[2] + acc[3])

        pltpu.sync_copy(out_v, out.at[pl.ds(base * _K, _BLK * _K)])
        return carry

    lax.fori_loop(0, _PW // _BLK, blk, 0)


_sc_gather = functools.partial(
    pl.kernel,
    mesh=plsc.VectorSubcoreMesh(core_axis_name="c", subcore_axis_name="s"),
    out_type=jax.ShapeDtypeStruct((_N * 16,), jnp.float32),
    scratch_types=[
        pltpu.VMEM((_BLK * _K,), jnp.int32),
        pltpu.VMEM((_BLK * _K,), jnp.float32),
        pltpu.VMEM((_BLK * _K, 16), jnp.float32),
        pltpu.VMEM((_BLK * 16,), jnp.float32),
        pltpu.SemaphoreType.DMA,
    ],
    compiler_params=pltpu.CompilerParams(use_tc_tiling_on_sc=False),
)(_sc_body)


def _lane_consts():
    w1 = np.zeros((128,), np.float32)
    w2 = np.zeros((128,), np.float32)
    ampm = np.zeros((128,), np.float32)
    s4 = np.zeros((128, 128), np.float32)
    c48 = np.zeros((128, 128), np.float32)
    a48 = np.zeros((128, 128), np.float32)
    passm = np.zeros((128,), np.float32)
    for l in range(128):
        c = l % 16
        u = l - c
        if c < 4:
            w1[l], w2[l], ampm[l] = 0.85, 0.15, 1.0
            a48[l, l + 4] = 1.0
            a48[l, l + 8] = 1.0
        elif c < 12:
            w1[l], w2[l] = 0.9, 0.1
            if c < 8:
                c48[l, l] = 1.0
                c48[l, l + 4] = 1.0
        else:
            w1[l] = 1.0
            if c in (12, 13):
                passm[l] = 1.0
        if c + 4 <= 15:
            s4[l + 4, l] = 1.0
    j = jnp.asarray
    return (j(w1)[None, :], j(w2)[None, :], j(ampm)[None, :],
            j(passm)[None, :], j(s4), j(c48), j(a48))


def _syn_body(t_ref, a_ref, x_ref, b_ref, s4_ref, c48_ref, a48_ref,
              w1_ref, w2_ref, am_ref, pm_ref, o_ref):
    hi = lax.Precision.HIGHEST
    dn = (((1,), (0,)), ((), ()))
    x = t_ref[...]
    a = a_ref[...]
    xx = x_ref[...]
    w1 = w1_ref[...]
    w2 = w2_ref[...]
    ampm = am_ref[...]
    passm = pm_ref[...]
    m = w1 * x + w2 * a
    mxa = lax.dot_general(xx, s4_ref[...], dn, precision=hi)
    sm = jnp.where(ampm > 0.0, jnp.clip(m, xx, mxa), m)
    sq = sm * sm
    qa = sq + lax.dot_general(sq, s4_ref[...], dn, precision=hi) + 1e-30
    inv = lax.rsqrt(qa)
    invb = lax.dot_general(inv, c48_ref[...], dn, precision=hi)
    smb = lax.dot_general(sm, a48_ref[...], dn, precision=hi)
    cp = smb * sm * invb + sm * passm
    ow = lax.dot_general(cp, b_ref[...], dn,
                         preferred_element_type=jnp.float32,
                         precision=lax.Precision.DEFAULT)
    o_ref[...] = ow.reshape(_BN, _T)


def kernel(time_vector, constant_offset, linear_trend, seasonal_amplitudes,
           seasonal_phases, neighbor_indices, neighbor_weights, min_bounds,
           max_bounds):
    f32 = jnp.float32
    ampf = seasonal_amplitudes.astype(f32).reshape(_PR, 128)
    phf = seasonal_phases.astype(f32).reshape(_PR, 128)
    mmf = jnp.concatenate(
        [min_bounds.astype(f32).reshape(_PR, 128),
         max_bounds.astype(f32).reshape(_PR, 128)], axis=1)
    otf = jnp.stack([constant_offset.astype(f32), linear_trend.astype(f32)],
                    axis=-1).reshape(_PR, 64)
    p1, p2, p3 = _perm_consts()

    tblp, auxp = pl.pallas_call(
        _prep_body,
        grid=(1,),
        in_specs=[
            pl.BlockSpec((_PB, 128), lambda i: (i, 0)),
            pl.BlockSpec((_PB, 128), lambda i: (i, 0)),
            pl.BlockSpec((_PB, 256), lambda i: (i, 0)),
            pl.BlockSpec((_PB, 64), lambda i: (i, 0)),
            pl.BlockSpec((384, 512), lambda i: (0, 0)),
            pl.BlockSpec((256, 512), lambda i: (0, 0)),
            pl.BlockSpec((64, 512), lambda i: (0, 0)),
        ],
        out_specs=[
            pl.BlockSpec((4 * _PB, 128), lambda i: (i, 0)),
            pl.BlockSpec((4 * _PB, 128), lambda i: (i, 0)),
        ],
        out_shape=[jax.ShapeDtypeStruct((_QR, 128), f32)] * 2,
    )(ampf, phf, mmf, otf, p1, p2, p3)

    idx_flat = neighbor_indices.astype(jnp.int32).reshape(_N * _K)
    w_flat = neighbor_weights.astype(f32).reshape(_N * _K)
    avgp = _sc_gather(tblp.reshape(_N, 16), idx_flat, w_flat).reshape(_QR, 128)

    tv = time_vector.astype(f32)
    freqs = jnp.array([4.0, 2.0, 1.0, 0.5], f32)
    ang = (2.0 * np.pi) * freqs[:, None] * tv[None, :]
    basis = jnp.concatenate(
        [jnp.zeros((4, _T), f32), jnp.sin(ang), jnp.cos(ang), tv[None, :],
         jnp.ones((1, _T), f32), jnp.zeros((2, _T), f32)], axis=0)
    bp = jnp.kron(jnp.eye(8, dtype=f32), basis)

    w1, w2, ampm, passm, s4, c48, a48 = _lane_consts()
    out = pl.pallas_call(
        _syn_body,
        grid=((_N + _BN - 1) // _BN,),
        in_specs=[
            pl.BlockSpec((_BQ, 128), lambda i: (i, 0)),
            pl.BlockSpec((_BQ, 128), lambda i: (i, 0)),
            pl.BlockSpec((_BQ, 128), lambda i: (i, 0)),
            pl.BlockSpec((128, 8 * _T), lambda i: (0, 0)),
            pl.BlockSpec((128, 128), lambda i: (0, 0)),
            pl.BlockSpec((128, 128), lambda i: (0, 0)),
            pl.BlockSpec((128, 128), lambda i: (0, 0)),
            pl.BlockSpec((1, 128), lambda i: (0, 0)),
            pl.BlockSpec((1, 128), lambda i: (0, 0)),
            pl.BlockSpec((1, 128), lambda i: (0, 0)),
            pl.BlockSpec((1, 128), lambda i: (0, 0)),
        ],
        out_specs=pl.BlockSpec((_BN, _T), lambda i: (i, 0)),
        out_shape=jax.ShapeDtypeStruct((_N, _T), f32),
    )(tblp, avgp, auxp, bp, s4, c48, a48, w1, w2, ampm, passm)
    return out
